# baseline (speedup 1.0000x reference)
import jax
import jax.numpy as jnp
from jax import lax
from jax.experimental import pallas as pl
from jax.experimental.pallas import tpu as pltpu
from jax.experimental.pallas import tpu_sc as plsc

_NC = 2
_NS = 16
_NW = _NC * _NS
_LANES = 16

_B = 4096
_L = 200
_V = 1000000
_D = 64
_CHUNK = 100
_GPR = _L // _CHUNK
_RING = 4

_TV = 512


def _transpose_body(t_ref, o_ref):
  tt = jnp.transpose(t_ref[...])
  d = tt.reshape(_TV // 2, 2, _D)
  o_ref[...] = jnp.concatenate([d[:, 0, :], d[:, 1, :]], axis=1)


def _tc_transpose(tableT):
  grid = (_V + _TV - 1) // _TV
  out = pl.pallas_call(
      _transpose_body,
      grid=(grid,),
      in_specs=[pl.BlockSpec((_D, _TV), lambda k: (0, k))],
      out_specs=pl.BlockSpec((_TV // 2, 2 * _D), lambda k: (k, 0)),
      out_shape=jax.ShapeDtypeStruct((_V // 2, 2 * _D), jnp.float32),
  )(tableT)
  return out.reshape(_V, _D)


def _sc_pool_body(table_hbm, idx_hbm, out_hbm, idx_all, bufs, pooled_v, sems):
  nb = _B // _NW
  ng = nb * _GPR
  wid = lax.axis_index("s") * _NC + lax.axis_index("c")
  base_i = wid * ng
  base_b = wid * nb

  pltpu.sync_copy(idx_hbm.at[pl.ds(base_i, ng)], idx_all)

  def fire(g, t):
    pltpu.async_copy(table_hbm.at[idx_all.at[g]], bufs.at[t], sems.at[t])

  for t in range(_RING):
    fire(t, t)

  inv_l = jnp.float32(1.0 / _L)

  def accum(buf, accs):
    def inner(i, accs):
      out = list(accs)
      for rr in range(4):
        r = i * 4 + rr
        for d in range(4):
          out[d] = out[d] + buf[r, pl.ds(d * _LANES, _LANES)]
      return tuple(out)
    return lax.fori_loop(0, _CHUNK // 4, inner, accs)

  def outer(j, carry):
    g0 = j * _RING
    accs = tuple(jnp.zeros((_LANES,), jnp.float32) for _ in range(4))
    for t in range(_RING):
      g = g0 + t
      pltpu.make_async_copy(
          table_hbm.at[idx_all.at[g0]], bufs.at[t], sems.at[t]).wait()
      accs = accum(bufs.at[t], accs)
      if t % _GPR == _GPR - 1:
        row = j * (_RING // _GPR) + t // _GPR
        for d in range(4):
          pooled_v[row, pl.ds(d * _LANES, _LANES)] = accs[d] * inv_l
        accs = tuple(jnp.zeros((_LANES,), jnp.float32) for _ in range(4))
      nxt = g + _RING

      @pl.when(nxt < ng)
      def _():
        fire(nxt, t)
    return carry

  lax.fori_loop(0, ng // _RING, outer, 0)
  pltpu.sync_copy(pooled_v, out_hbm.at[pl.ds(base_b, nb)])


def _sc_pool(table, idx2):
  nb = _B // _NW
  ng = nb * _GPR
  mesh = plsc.VectorSubcoreMesh(core_axis_name="c", subcore_axis_name="s")
  return pl.kernel(
      _sc_pool_body,
      out_type=jax.ShapeDtypeStruct((_B, _D), jnp.float32),
      mesh=mesh,
      compiler_params=pltpu.CompilerParams(use_tc_tiling_on_sc=False),
      scratch_types=[
          pltpu.VMEM((ng, _CHUNK), jnp.int32),
          pltpu.VMEM((_RING, _CHUNK, _D), jnp.float32),
          pltpu.VMEM((nb, _D), jnp.float32),
          pltpu.SemaphoreType.DMA((_RING,)),
      ],
  )(table, idx2)


def _mlp_body(p_ref, w1_ref, b1_ref, w2_ref, b2_ref, w3_ref, b3_ref, o_ref):
  h = jnp.dot(p_ref[...], w1_ref[...], preferred_element_type=jnp.float32)
  h = jnp.maximum(h + b1_ref[...], 0.0)
  h = jnp.dot(h, w2_ref[...], preferred_element_type=jnp.float32)
  h = jnp.maximum(h + b2_ref[...], 0.0)
  o_ref[...] = (
      jnp.dot(h, w3_ref[...], preferred_element_type=jnp.float32)
      + b3_ref[...])


def _mlp(pooled, W1, b1, W2, b2, W3, b3):
  return pl.pallas_call(
      _mlp_body,
      out_shape=jax.ShapeDtypeStruct((pooled.shape[0], W3.shape[1]),
                                     jnp.float32),
  )(pooled, W1, b1.reshape(1, -1), W2, b2.reshape(1, -1),
    W3, b3.reshape(1, -1))


def kernel(x, table, W1, b1, W2, b2, W3, b3):
  table_lin = _tc_transpose(table.T)
  idx2 = x.reshape(_B * _GPR, _CHUNK).astype(jnp.int32)
  pooled = _sc_pool(table_lin, idx2)
  return _mlp(pooled, W1, b1, W2, b2, W3, b3)


# speedup vs baseline: 1.0258x; 1.0258x over previous
import jax
import jax.numpy as jnp
from jax import lax
from jax.experimental import pallas as pl
from jax.experimental.pallas import tpu as pltpu
from jax.experimental.pallas import tpu_sc as plsc

_NC = 2
_NS = 16
_NW = _NC * _NS
_LANES = 16

_B = 4096
_L = 200
_V = 1000000
_D = 64
_CHUNK = 100
_GPR = _L // _CHUNK
_RING = 4

_TV = 512
_NBLK = (_V + _TV - 1) // _TV
_VPAD = _NBLK * _TV


def _relayout_body(t_ref, o_ref):
  t = t_ref[...]
  lo = jnp.transpose(t[:, 0:_TV // 2])
  hi = jnp.transpose(t[:, _TV // 2:_TV])
  o_ref[...] = jnp.concatenate([lo, hi], axis=1)


def _tc_relayout(tableT):
  out = pl.pallas_call(
      _relayout_body,
      grid=(_NBLK,),
      in_specs=[pl.BlockSpec((_D, _TV), lambda k: (0, k))],
      out_specs=pl.BlockSpec((_TV // 2, 2 * _D), lambda k: (k, 0)),
      out_shape=jax.ShapeDtypeStruct((_VPAD // 2, 2 * _D), jnp.float32),
  )(tableT)
  return out.reshape(_VPAD, _D)


def _sc_pool_body(table_hbm, idx_hbm, out_hbm, idx_all, bufs, pooled_v, sems):
  nb = _B // _NW
  ng = nb * _GPR
  wid = lax.axis_index("s") * _NC + lax.axis_index("c")
  base_i = wid * ng
  base_b = wid * nb

  pltpu.sync_copy(idx_hbm.at[pl.ds(base_i, ng)], idx_all)

  def remap(v):
    blk = jnp.bitwise_and(v, jnp.int32(~511))
    lo = lax.shift_left(jnp.bitwise_and(v, 255), 1)
    hi = lax.shift_right_logical(jnp.bitwise_and(v, 511), 8)
    return jnp.bitwise_or(jnp.bitwise_or(blk, lo), hi)

  tail_col = _CHUNK - _LANES

  def split(i, carry):
    def one(k, _):
      r = i * 4 + k
      vtail = idx_all[r, pl.ds(tail_col, _LANES)]
      for c in range(_CHUNK // _LANES):
        col = c * _LANES
        idx_all[r, pl.ds(col, _LANES)] = remap(
            idx_all[r, pl.ds(col, _LANES)])
      idx_all[r, pl.ds(tail_col, _LANES)] = remap(vtail)
      return 0
    return lax.fori_loop(0, 4, one, 0)

  lax.fori_loop(0, ng // 4, split, 0)

  def fire(g, t):
    pltpu.async_copy(table_hbm.at[idx_all.at[g]], bufs.at[t], sems.at[t])

  for t in range(_RING):
    fire(t, t)

  inv_l = jnp.float32(1.0 / _L)

  def accum(buf, accs):
    def inner(i, accs):
      out = list(accs)
      for rr in range(4):
        r = i * 4 + rr
        for d in range(4):
          out[d] = out[d] + buf[r, pl.ds(d * _LANES, _LANES)]
      return tuple(out)
    return lax.fori_loop(0, _CHUNK // 4, inner, accs)

  def outer(j, carry):
    g0 = j * _RING
    accs = tuple(jnp.zeros((_LANES,), jnp.float32) for _ in range(4))
    for t in range(_RING):
      g = g0 + t
      pltpu.make_async_copy(
          table_hbm.at[idx_all.at[g0]], bufs.at[t], sems.at[t]).wait()
      accs = accum(bufs.at[t], accs)
      if t % _GPR == _GPR - 1:
        row = j * (_RING // _GPR) + t // _GPR
        for d in range(4):
          pooled_v[row, pl.ds(d * _LANES, _LANES)] = accs[d] * inv_l
        accs = tuple(jnp.zeros((_LANES,), jnp.float32) for _ in range(4))
      nxt = g + _RING

      @pl.when(nxt < ng)
      def _():
        fire(nxt, t)
    return carry

  lax.fori_loop(0, ng // _RING, outer, 0)
  pltpu.sync_copy(pooled_v, out_hbm.at[pl.ds(base_b, nb)])


def _sc_pool(table_lin, idx2):
  nb = _B // _NW
  ng = nb * _GPR
  mesh = plsc.VectorSubcoreMesh(core_axis_name="c", subcore_axis_name="s")
  return pl.kernel(
      _sc_pool_body,
      out_type=jax.ShapeDtypeStruct((_B, _D), jnp.float32),
      mesh=mesh,
      compiler_params=pltpu.CompilerParams(use_tc_tiling_on_sc=False),
      scratch_types=[
          pltpu.VMEM((ng, _CHUNK), jnp.int32),
          pltpu.VMEM((_RING, _CHUNK, _D), jnp.float32),
          pltpu.VMEM((nb, _D), jnp.float32),
          pltpu.SemaphoreType.DMA((_RING,)),
      ],
  )(table_lin, idx2)


def _mlp_body(p_ref, w1_ref, b1_ref, w2_ref, b2_ref, w3_ref, b3_ref, o_ref):
  h = jnp.dot(p_ref[...], w1_ref[...], preferred_element_type=jnp.float32)
  h = jnp.maximum(h + b1_ref[...], 0.0)
  h = jnp.dot(h, w2_ref[...], preferred_element_type=jnp.float32)
  h = jnp.maximum(h + b2_ref[...], 0.0)
  o_ref[...] = (
      jnp.dot(h, w3_ref[...], preferred_element_type=jnp.float32)
      + b3_ref[...])


def _mlp(pooled, W1, b1, W2, b2, W3, b3):
  return pl.pallas_call(
      _mlp_body,
      out_shape=jax.ShapeDtypeStruct((pooled.shape[0], W3.shape[1]),
                                     jnp.float32),
  )(pooled, W1, b1.reshape(1, -1), W2, b2.reshape(1, -1),
    W3, b3.reshape(1, -1))


def kernel(x, table, W1, b1, W2, b2, W3, b3):
  table_lin = _tc_relayout(table.T)
  idx2 = x.reshape(_B * _GPR, _CHUNK).astype(jnp.int32)
  pooled = _sc_pool(table_lin, idx2)
  return _mlp(pooled, W1, b1, W2, b2, W3, b3)


# speedup vs baseline: 2.9941x; 2.9187x over previous
import jax
import jax.numpy as jnp
from jax import lax
from jax.experimental import pallas as pl
from jax.experimental.pallas import tpu as pltpu
from jax.experimental.pallas import tpu_sc as plsc

_NC = 2
_NS = 16
_NW = _NC * _NS
_LANES = 16

_B = 4096
_L = 200
_V = 1000000
_D = 64
_CHUNK = 100
_GPR = _L // _CHUNK
_RING = 4

_TV = 4096
_NBLK = (_V + _TV - 1) // _TV
_VPAD = _NBLK * _TV


def _relayout_body(t_ref, o_ref):
  t = t_ref[...]
  lo = jnp.transpose(t[:, 0:_TV // 2])
  hi = jnp.transpose(t[:, _TV // 2:_TV])
  o_ref[...] = jnp.concatenate([lo, hi], axis=1)


def _tc_relayout(tableT):
  out = pl.pallas_call(
      _relayout_body,
      grid=(_NBLK,),
      in_specs=[pl.BlockSpec((_D, _TV), lambda k: (0, k))],
      out_specs=pl.BlockSpec((_TV // 2, 2 * _D), lambda k: (k, 0)),
      out_shape=jax.ShapeDtypeStruct((_VPAD // 2, 2 * _D), jnp.float32),
  )(tableT)
  return out.reshape(_VPAD, _D)


def _sc_pool_body(table_hbm, idx_hbm, out_hbm, idx_all, bufs, pooled_v, sems):
  nb = _B // _NW
  ng = nb * _GPR
  wid = lax.axis_index("s") * _NC + lax.axis_index("c")
  base_i = wid * ng
  base_b = wid * nb

  pltpu.sync_copy(idx_hbm.at[pl.ds(base_i, ng)], idx_all)

  def remap(v):
    blk = jnp.bitwise_and(v, jnp.int32(~(_TV - 1)))
    lo = lax.shift_left(jnp.bitwise_and(v, _TV // 2 - 1), 1)
    hi = lax.shift_right_logical(jnp.bitwise_and(v, _TV - 1),
                                 (_TV // 2).bit_length() - 1)
    return jnp.bitwise_or(jnp.bitwise_or(blk, lo), hi)

  tail_col = _CHUNK - _LANES

  def split(i, carry):
    def one(k, _):
      r = i * 4 + k
      vtail = idx_all[r, pl.ds(tail_col, _LANES)]
      for c in range(_CHUNK // _LANES):
        col = c * _LANES
        idx_all[r, pl.ds(col, _LANES)] = remap(
            idx_all[r, pl.ds(col, _LANES)])
      idx_all[r, pl.ds(tail_col, _LANES)] = remap(vtail)
      return 0
    return lax.fori_loop(0, 4, one, 0)

  lax.fori_loop(0, ng // 4, split, 0)

  def fire(g, t):
    pltpu.async_copy(table_hbm.at[idx_all.at[g]], bufs.at[t], sems.at[t])

  for t in range(_RING):
    fire(t, t)

  inv_l = jnp.float32(1.0 / _L)

  def accum(buf, accs):
    def inner(i, accs):
      out = list(accs)
      for rr in range(4):
        r = i * 4 + rr
        for d in range(4):
          out[d] = out[d] + buf[r, pl.ds(d * _LANES, _LANES)]
      return tuple(out)
    return lax.fori_loop(0, _CHUNK // 4, inner, accs)

  def outer(j, carry):
    g0 = j * _RING
    accs = tuple(jnp.zeros((_LANES,), jnp.float32) for _ in range(4))
    for t in range(_RING):
      g = g0 + t
      pltpu.make_async_copy(
          table_hbm.at[idx_all.at[g0]], bufs.at[t], sems.at[t]).wait()
      accs = accum(bufs.at[t], accs)
      if t % _GPR == _GPR - 1:
        row = j * (_RING // _GPR) + t // _GPR
        for d in range(4):
          pooled_v[row, pl.ds(d * _LANES, _LANES)] = accs[d] * inv_l
        accs = tuple(jnp.zeros((_LANES,), jnp.float32) for _ in range(4))
      nxt = g + _RING

      @pl.when(nxt < ng)
      def _():
        fire(nxt, t)
    return carry

  lax.fori_loop(0, ng // _RING, outer, 0)
  pltpu.sync_copy(pooled_v, out_hbm.at[pl.ds(base_b, nb)])


def _sc_pool(table_lin, idx2):
  nb = _B // _NW
  ng = nb * _GPR
  mesh = plsc.VectorSubcoreMesh(core_axis_name="c", subcore_axis_name="s")
  return pl.kernel(
      _sc_pool_body,
      out_type=jax.ShapeDtypeStruct((_B, _D), jnp.float32),
      mesh=mesh,
      compiler_params=pltpu.CompilerParams(use_tc_tiling_on_sc=False),
      scratch_types=[
          pltpu.VMEM((ng, _CHUNK), jnp.int32),
          pltpu.VMEM((_RING, _CHUNK, _D), jnp.float32),
          pltpu.VMEM((nb, _D), jnp.float32),
          pltpu.SemaphoreType.DMA((_RING,)),
      ],
  )(table_lin, idx2)


def _mlp_body(p_ref, w1_ref, b1_ref, w2_ref, b2_ref, w3_ref, b3_ref, o_ref):
  h = jnp.dot(p_ref[...], w1_ref[...], preferred_element_type=jnp.float32)
  h = jnp.maximum(h + b1_ref[...], 0.0)
  h = jnp.dot(h, w2_ref[...], preferred_element_type=jnp.float32)
  h = jnp.maximum(h + b2_ref[...], 0.0)
  o_ref[...] = (
      jnp.dot(h, w3_ref[...], preferred_element_type=jnp.float32)
      + b3_ref[...])


def _mlp(pooled, W1, b1, W2, b2, W3, b3):
  return pl.pallas_call(
      _mlp_body,
      out_shape=jax.ShapeDtypeStruct((pooled.shape[0], W3.shape[1]),
                                     jnp.float32),
  )(pooled, W1, b1.reshape(1, -1), W2, b2.reshape(1, -1),
    W3, b3.reshape(1, -1))


def kernel(x, table, W1, b1, W2, b2, W3, b3):
  table_lin = _tc_relayout(table.T)
  idx2 = x.reshape(_B * _GPR, _CHUNK).astype(jnp.int32)
  pooled = _sc_pool(table_lin, idx2)
  return _mlp(pooled, W1, b1, W2, b2, W3, b3)


# speedup vs baseline: 3.5052x; 1.1707x over previous
import jax
import jax.numpy as jnp
from jax import lax
from jax.experimental import pallas as pl
from jax.experimental.pallas import tpu as pltpu
from jax.experimental.pallas import tpu_sc as plsc

_NC = 2
_NS = 16
_NW = _NC * _NS
_LANES = 16

_B = 4096
_L = 200
_V = 1000000
_D = 64
_CHUNK = 100
_GPR = _L // _CHUNK
_RING = 4

_TV = 8192
_NBLK = (_V + _TV - 1) // _TV
_VPAD = _NBLK * _TV


def _relayout_body(t_ref, o_ref):
  t = t_ref[...]
  lo = jnp.transpose(t[:, 0:_TV // 2])
  hi = jnp.transpose(t[:, _TV // 2:_TV])
  o_ref[...] = jnp.concatenate([lo, hi], axis=1)


def _tc_relayout(tableT):
  out = pl.pallas_call(
      _relayout_body,
      grid=(_NBLK,),
      in_specs=[pl.BlockSpec((_D, _TV), lambda k: (0, k))],
      out_specs=pl.BlockSpec((_TV // 2, 2 * _D), lambda k: (k, 0)),
      out_shape=jax.ShapeDtypeStruct((_VPAD // 2, 2 * _D), jnp.float32),
  )(tableT)
  return out.reshape(_VPAD, _D)


def _sc_pool_body(table_hbm, idx_hbm, out_hbm, idx_all, bufs, pooled_v, sems):
  nb = _B // _NW
  ng = nb * _GPR
  wid = lax.axis_index("s") * _NC + lax.axis_index("c")
  base_i = wid * ng
  base_b = wid * nb

  pltpu.sync_copy(idx_hbm.at[pl.ds(base_i, ng)], idx_all)

  def remap(v):
    blk = jnp.bitwise_and(v, jnp.int32(~(_TV - 1)))
    lo = lax.shift_left(jnp.bitwise_and(v, _TV // 2 - 1), 1)
    hi = lax.shift_right_logical(jnp.bitwise_and(v, _TV - 1),
                                 (_TV // 2).bit_length() - 1)
    return jnp.bitwise_or(jnp.bitwise_or(blk, lo), hi)

  tail_col = _CHUNK - _LANES

  def split(i, carry):
    def one(k, _):
      r = i * 4 + k
      vtail = idx_all[r, pl.ds(tail_col, _LANES)]
      for c in range(_CHUNK // _LANES):
        col = c * _LANES
        idx_all[r, pl.ds(col, _LANES)] = remap(
            idx_all[r, pl.ds(col, _LANES)])
      idx_all[r, pl.ds(tail_col, _LANES)] = remap(vtail)
      return 0
    return lax.fori_loop(0, 4, one, 0)

  lax.fori_loop(0, ng // 4, split, 0)

  def fire(g, t):
    pltpu.async_copy(table_hbm.at[idx_all.at[g]], bufs.at[t], sems.at[t])

  for t in range(_RING):
    fire(t, t)

  inv_l = jnp.float32(1.0 / _L)

  def accum(buf, accs):
    def inner(i, accs):
      out = list(accs)
      for rr in range(4):
        r = i * 4 + rr
        for d in range(4):
          out[d] = out[d] + buf[r, pl.ds(d * _LANES, _LANES)]
      return tuple(out)
    return lax.fori_loop(0, _CHUNK // 4, inner, accs)

  def outer(j, carry):
    g0 = j * _RING
    accs = tuple(jnp.zeros((_LANES,), jnp.float32) for _ in range(4))
    for t in range(_RING):
      g = g0 + t
      pltpu.make_async_copy(
          table_hbm.at[idx_all.at[g0]], bufs.at[t], sems.at[t]).wait()
      accs = accum(bufs.at[t], accs)
      if t % _GPR == _GPR - 1:
        row = j * (_RING // _GPR) + t // _GPR
        for d in range(4):
          pooled_v[row, pl.ds(d * _LANES, _LANES)] = accs[d] * inv_l
        accs = tuple(jnp.zeros((_LANES,), jnp.float32) for _ in range(4))
      nxt = g + _RING

      @pl.when(nxt < ng)
      def _():
        fire(nxt, t)
    return carry

  lax.fori_loop(0, ng // _RING, outer, 0)
  pltpu.sync_copy(pooled_v, out_hbm.at[pl.ds(base_b, nb)])


def _sc_pool(table_lin, idx2):
  nb = _B // _NW
  ng = nb * _GPR
  mesh = plsc.VectorSubcoreMesh(core_axis_name="c", subcore_axis_name="s")
  return pl.kernel(
      _sc_pool_body,
      out_type=jax.ShapeDtypeStruct((_B, _D), jnp.float32),
      mesh=mesh,
      compiler_params=pltpu.CompilerParams(use_tc_tiling_on_sc=False),
      scratch_types=[
          pltpu.VMEM((ng, _CHUNK), jnp.int32),
          pltpu.VMEM((_RING, _CHUNK, _D), jnp.float32),
          pltpu.VMEM((nb, _D), jnp.float32),
          pltpu.SemaphoreType.DMA((_RING,)),
      ],
  )(table_lin, idx2)


def _mlp_body(p_ref, w1_ref, b1_ref, w2_ref, b2_ref, w3_ref, b3_ref, o_ref):
  h = jnp.dot(p_ref[...], w1_ref[...], preferred_element_type=jnp.float32)
  h = jnp.maximum(h + b1_ref[...], 0.0)
  h = jnp.dot(h, w2_ref[...], preferred_element_type=jnp.float32)
  h = jnp.maximum(h + b2_ref[...], 0.0)
  o_ref[...] = (
      jnp.dot(h, w3_ref[...], preferred_element_type=jnp.float32)
      + b3_ref[...])


def _mlp(pooled, W1, b1, W2, b2, W3, b3):
  return pl.pallas_call(
      _mlp_body,
      out_shape=jax.ShapeDtypeStruct((pooled.shape[0], W3.shape[1]),
                                     jnp.float32),
  )(pooled, W1, b1.reshape(1, -1), W2, b2.reshape(1, -1),
    W3, b3.reshape(1, -1))


def kernel(x, table, W1, b1, W2, b2, W3, b3):
  table_lin = _tc_relayout(table.T)
  idx2 = x.reshape(_B * _GPR, _CHUNK).astype(jnp.int32)
  pooled = _sc_pool(table_lin, idx2)
  return _mlp(pooled, W1, b1, W2, b2, W3, b3)


# speedup vs baseline: 3.8295x; 1.0925x over previous
import jax
import jax.numpy as jnp
from jax import lax
from jax.experimental import pallas as pl
from jax.experimental.pallas import tpu as pltpu
from jax.experimental.pallas import tpu_sc as plsc

_NC = 2
_NS = 16
_NW = _NC * _NS
_LANES = 16

_B = 4096
_L = 200
_V = 1000000
_D = 64
_CHUNK = 100
_GPR = _L // _CHUNK
_RING = 4

_TV = 16384
_NBLK = (_V + _TV - 1) // _TV
_VPAD = _NBLK * _TV


def _relayout_body(t_ref, o_ref):
  t = t_ref[...]
  lo = jnp.transpose(t[:, 0:_TV // 2])
  hi = jnp.transpose(t[:, _TV // 2:_TV])
  o_ref[...] = jnp.concatenate([lo, hi], axis=1)


def _tc_relayout(tableT):
  out = pl.pallas_call(
      _relayout_body,
      grid=(_NBLK,),
      in_specs=[pl.BlockSpec((_D, _TV), lambda k: (0, k))],
      out_specs=pl.BlockSpec((_TV // 2, 2 * _D), lambda k: (k, 0)),
      out_shape=jax.ShapeDtypeStruct((_VPAD // 2, 2 * _D), jnp.float32),
  )(tableT)
  return out.reshape(_VPAD, _D)


def _sc_pool_body(table_hbm, idx_hbm, out_hbm, idx_all, bufs, pooled_v, sems):
  nb = _B // _NW
  ng = nb * _GPR
  wid = lax.axis_index("s") * _NC + lax.axis_index("c")
  base_i = wid * ng
  base_b = wid * nb

  pltpu.sync_copy(idx_hbm.at[pl.ds(base_i, ng)], idx_all)

  def remap(v):
    blk = jnp.bitwise_and(v, jnp.int32(~(_TV - 1)))
    lo = lax.shift_left(jnp.bitwise_and(v, _TV // 2 - 1), 1)
    hi = lax.shift_right_logical(jnp.bitwise_and(v, _TV - 1),
                                 (_TV // 2).bit_length() - 1)
    return jnp.bitwise_or(jnp.bitwise_or(blk, lo), hi)

  tail_col = _CHUNK - _LANES

  def split(i, carry):
    def one(k, _):
      r = i * 4 + k
      vtail = idx_all[r, pl.ds(tail_col, _LANES)]
      for c in range(_CHUNK // _LANES):
        col = c * _LANES
        idx_all[r, pl.ds(col, _LANES)] = remap(
            idx_all[r, pl.ds(col, _LANES)])
      idx_all[r, pl.ds(tail_col, _LANES)] = remap(vtail)
      return 0
    return lax.fori_loop(0, 4, one, 0)

  lax.fori_loop(0, ng // 4, split, 0)

  def fire(g, t):
    pltpu.async_copy(table_hbm.at[idx_all.at[g]], bufs.at[t], sems.at[t])

  for t in range(_RING):
    fire(t, t)

  inv_l = jnp.float32(1.0 / _L)

  def accum(buf, accs):
    def inner(i, accs):
      out = list(accs)
      for rr in range(4):
        r = i * 4 + rr
        for d in range(4):
          out[d] = out[d] + buf[r, pl.ds(d * _LANES, _LANES)]
      return tuple(out)
    return lax.fori_loop(0, _CHUNK // 4, inner, accs)

  def outer(j, carry):
    g0 = j * _RING
    accs = tuple(jnp.zeros((_LANES,), jnp.float32) for _ in range(4))
    for t in range(_RING):
      g = g0 + t
      pltpu.make_async_copy(
          table_hbm.at[idx_all.at[g0]], bufs.at[t], sems.at[t]).wait()
      accs = accum(bufs.at[t], accs)
      if t % _GPR == _GPR - 1:
        row = j * (_RING // _GPR) + t // _GPR
        for d in range(4):
          pooled_v[row, pl.ds(d * _LANES, _LANES)] = accs[d] * inv_l
        accs = tuple(jnp.zeros((_LANES,), jnp.float32) for _ in range(4))
      nxt = g + _RING

      @pl.when(nxt < ng)
      def _():
        fire(nxt, t)
    return carry

  lax.fori_loop(0, ng // _RING, outer, 0)
  pltpu.sync_copy(pooled_v, out_hbm.at[pl.ds(base_b, nb)])


def _sc_pool(table_lin, idx2):
  nb = _B // _NW
  ng = nb * _GPR
  mesh = plsc.VectorSubcoreMesh(core_axis_name="c", subcore_axis_name="s")
  return pl.kernel(
      _sc_pool_body,
      out_type=jax.ShapeDtypeStruct((_B, _D), jnp.float32),
      mesh=mesh,
      compiler_params=pltpu.CompilerParams(use_tc_tiling_on_sc=False),
      scratch_types=[
          pltpu.VMEM((ng, _CHUNK), jnp.int32),
          pltpu.VMEM((_RING, _CHUNK, _D), jnp.float32),
          pltpu.VMEM((nb, _D), jnp.float32),
          pltpu.SemaphoreType.DMA((_RING,)),
      ],
  )(table_lin, idx2)


def _mlp_body(p_ref, w1_ref, b1_ref, w2_ref, b2_ref, w3_ref, b3_ref, o_ref):
  h = jnp.dot(p_ref[...], w1_ref[...], preferred_element_type=jnp.float32)
  h = jnp.maximum(h + b1_ref[...], 0.0)
  h = jnp.dot(h, w2_ref[...], preferred_element_type=jnp.float32)
  h = jnp.maximum(h + b2_ref[...], 0.0)
  o_ref[...] = (
      jnp.dot(h, w3_ref[...], preferred_element_type=jnp.float32)
      + b3_ref[...])


def _mlp(pooled, W1, b1, W2, b2, W3, b3):
  return pl.pallas_call(
      _mlp_body,
      out_shape=jax.ShapeDtypeStruct((pooled.shape[0], W3.shape[1]),
                                     jnp.float32),
  )(pooled, W1, b1.reshape(1, -1), W2, b2.reshape(1, -1),
    W3, b3.reshape(1, -1))


def kernel(x, table, W1, b1, W2, b2, W3, b3):
  table_lin = _tc_relayout(table.T)
  idx2 = x.reshape(_B * _GPR, _CHUNK).astype(jnp.int32)
  pooled = _sc_pool(table_lin, idx2)
  return _mlp(pooled, W1, b1, W2, b2, W3, b3)


# speedup vs baseline: 3.9945x; 1.0431x over previous
import jax
import jax.numpy as jnp
from jax import lax
from jax.experimental import pallas as pl
from jax.experimental.pallas import tpu as pltpu
from jax.experimental.pallas import tpu_sc as plsc

_NC = 2
_NS = 16
_NW = _NC * _NS
_LANES = 16

_B = 4096
_L = 200
_V = 1000000
_D = 64
_CHUNK = 100
_GPR = _L // _CHUNK
_RING = 4

_TV = 32768
_NBLK = (_V + _TV - 1) // _TV
_VPAD = _NBLK * _TV


def _relayout_body(t_ref, o_ref):
  t = t_ref[...]
  lo = jnp.transpose(t[:, 0:_TV // 2])
  hi = jnp.transpose(t[:, _TV // 2:_TV])
  o_ref[...] = jnp.concatenate([lo, hi], axis=1)


def _tc_relayout(tableT):
  out = pl.pallas_call(
      _relayout_body,
      grid=(_NBLK,),
      in_specs=[pl.BlockSpec((_D, _TV), lambda k: (0, k))],
      out_specs=pl.BlockSpec((_TV // 2, 2 * _D), lambda k: (k, 0)),
      out_shape=jax.ShapeDtypeStruct((_VPAD // 2, 2 * _D), jnp.float32),
  )(tableT)
  return out.reshape(_VPAD, _D)


def _sc_pool_body(table_hbm, idx_hbm, out_hbm, idx_all, bufs, pooled_v, sems):
  nb = _B // _NW
  ng = nb * _GPR
  wid = lax.axis_index("s") * _NC + lax.axis_index("c")
  base_i = wid * ng
  base_b = wid * nb

  pltpu.sync_copy(idx_hbm.at[pl.ds(base_i, ng)], idx_all)

  def remap(v):
    blk = jnp.bitwise_and(v, jnp.int32(~(_TV - 1)))
    lo = lax.shift_left(jnp.bitwise_and(v, _TV // 2 - 1), 1)
    hi = lax.shift_right_logical(jnp.bitwise_and(v, _TV - 1),
                                 (_TV // 2).bit_length() - 1)
    return jnp.bitwise_or(jnp.bitwise_or(blk, lo), hi)

  tail_col = _CHUNK - _LANES

  def split(i, carry):
    def one(k, _):
      r = i * 4 + k
      vtail = idx_all[r, pl.ds(tail_col, _LANES)]
      for c in range(_CHUNK // _LANES):
        col = c * _LANES
        idx_all[r, pl.ds(col, _LANES)] = remap(
            idx_all[r, pl.ds(col, _LANES)])
      idx_all[r, pl.ds(tail_col, _LANES)] = remap(vtail)
      return 0
    return lax.fori_loop(0, 4, one, 0)

  lax.fori_loop(0, ng // 4, split, 0)

  def fire(g, t):
    pltpu.async_copy(table_hbm.at[idx_all.at[g]], bufs.at[t], sems.at[t])

  for t in range(_RING):
    fire(t, t)

  inv_l = jnp.float32(1.0 / _L)

  def accum(buf, accs):
    def inner(i, accs):
      out = list(accs)
      for rr in range(4):
        r = i * 4 + rr
        for d in range(4):
          out[d] = out[d] + buf[r, pl.ds(d * _LANES, _LANES)]
      return tuple(out)
    return lax.fori_loop(0, _CHUNK // 4, inner, accs)

  def outer(j, carry):
    g0 = j * _RING
    accs = tuple(jnp.zeros((_LANES,), jnp.float32) for _ in range(4))
    for t in range(_RING):
      g = g0 + t
      pltpu.make_async_copy(
          table_hbm.at[idx_all.at[g0]], bufs.at[t], sems.at[t]).wait()
      accs = accum(bufs.at[t], accs)
      if t % _GPR == _GPR - 1:
        row = j * (_RING // _GPR) + t // _GPR
        for d in range(4):
          pooled_v[row, pl.ds(d * _LANES, _LANES)] = accs[d] * inv_l
        accs = tuple(jnp.zeros((_LANES,), jnp.float32) for _ in range(4))
      nxt = g + _RING

      @pl.when(nxt < ng)
      def _():
        fire(nxt, t)
    return carry

  lax.fori_loop(0, ng // _RING, outer, 0)
  pltpu.sync_copy(pooled_v, out_hbm.at[pl.ds(base_b, nb)])


def _sc_pool(table_lin, idx2):
  nb = _B // _NW
  ng = nb * _GPR
  mesh = plsc.VectorSubcoreMesh(core_axis_name="c", subcore_axis_name="s")
  return pl.kernel(
      _sc_pool_body,
      out_type=jax.ShapeDtypeStruct((_B, _D), jnp.float32),
      mesh=mesh,
      compiler_params=pltpu.CompilerParams(use_tc_tiling_on_sc=False),
      scratch_types=[
          pltpu.VMEM((ng, _CHUNK), jnp.int32),
          pltpu.VMEM((_RING, _CHUNK, _D), jnp.float32),
          pltpu.VMEM((nb, _D), jnp.float32),
          pltpu.SemaphoreType.DMA((_RING,)),
      ],
  )(table_lin, idx2)


def _mlp_body(p_ref, w1_ref, b1_ref, w2_ref, b2_ref, w3_ref, b3_ref, o_ref):
  h = jnp.dot(p_ref[...], w1_ref[...], preferred_element_type=jnp.float32)
  h = jnp.maximum(h + b1_ref[...], 0.0)
  h = jnp.dot(h, w2_ref[...], preferred_element_type=jnp.float32)
  h = jnp.maximum(h + b2_ref[...], 0.0)
  o_ref[...] = (
      jnp.dot(h, w3_ref[...], preferred_element_type=jnp.float32)
      + b3_ref[...])


def _mlp(pooled, W1, b1, W2, b2, W3, b3):
  return pl.pallas_call(
      _mlp_body,
      out_shape=jax.ShapeDtypeStruct((pooled.shape[0], W3.shape[1]),
                                     jnp.float32),
  )(pooled, W1, b1.reshape(1, -1), W2, b2.reshape(1, -1),
    W3, b3.reshape(1, -1))


def kernel(x, table, W1, b1, W2, b2, W3, b3):
  table_lin = _tc_relayout(table.T)
  idx2 = x.reshape(_B * _GPR, _CHUNK).astype(jnp.int32)
  pooled = _sc_pool(table_lin, idx2)
  return _mlp(pooled, W1, b1, W2, b2, W3, b3)
